# Initial kernel scaffold; baseline (speedup 1.0000x reference)
import jax
import jax.numpy as jnp
from jax.experimental import pallas as pl


def kernel(voxel_features, coords, batch_size):
    raise NotImplementedError("write your pallas kernel here")


# speedup vs baseline: 5.5074x; 5.5074x over previous
import functools

import jax
import jax.numpy as jnp
from jax import lax
from jax.experimental import pallas as pl
from jax.experimental.pallas import tpu as pltpu
from jax.experimental.pallas import tpu_sc as plsc

C = 64
NX = 512
NY = 512
NB = 4
NCELL = NX * NY
PLANE = C * NCELL
TOTAL = NB * PLANE
L = 16
NSUB = 16
CELL_BITS = 24
LCAP = 4096
SCAP = 288
STRIP = 256
RCELLS = 8192


def _zero_body(o_ref):
    o_ref[...] = jnp.zeros_like(o_ref)


def _make_canvas():
    return pl.pallas_call(
        _zero_body,
        out_shape=jax.ShapeDtypeStruct((TOTAL,), jnp.float32),
        grid=(32,),
        out_specs=pl.BlockSpec((TOTAL // 32,), lambda i: (i,)),
    )()


def _sc_scatter_body(canvas, coords_h, feats_h, bound_h,
                     coords_v, bound_v, lbase, lp, cnt16, counts_v, wmap,
                     strip_base, strip_p, base_sc, p_sc, rows_v, idxbuf,
                     cnt_spm, spm_base, spm_p, P, CH, G):
    cidx = lax.axis_index("c")
    sidx = lax.axis_index("s")
    iota = lax.broadcasted_iota(jnp.int32, (L,), 0)
    zeros16 = jnp.zeros((L,), jnp.int32)
    rid = cidx * NSUB + sidx
    canvas_flat = canvas
    coff = [(cb * L + iota) * NCELL for cb in range(C // L)]

    pltpu.sync_copy(bound_h, bound_v)
    pltpu.sync_copy(coords_h.at[pl.ds(sidx * CH * 4, CH * 4)], coords_v)
    bvec = bound_v[...]

    def scan_body(g, cnt):
        pidx = g * L + iota
        valid = (pidx < CH) & ((sidx * CH + pidx) < P)
        b = plsc.load_gather(coords_v, [pidx * 4], mask=valid)
        x = plsc.load_gather(coords_v, [pidx * 4 + 1], mask=valid)
        y = plsc.load_gather(coords_v, [pidx * 4 + 2], mask=valid)
        cell = x * NY + y
        acc = valid & (b >= 0) & (b < bvec) & ((cell >> 17) == cidx)
        base = (b << CELL_BITS) | cell
        pg = sidx * CH + pidx
        plsc.store_compressed(lbase.at[pl.ds(cnt, L)], base, mask=acc)
        plsc.store_compressed(lp.at[pl.ds(cnt, L)], pg, mask=acc)
        return cnt + jnp.sum(acc.astype(jnp.int32))

    cnt = lax.fori_loop(0, G, scan_body, jnp.int32(0))

    cnt16[...] = jnp.full((L,), cnt, jnp.int32)
    pltpu.sync_copy(cnt16, cnt_spm.at[pl.ds(sidx * L, L)])

    def pub(k, _):
        pltpu.sync_copy(lbase.at[pl.ds(k * STRIP, STRIP)],
                        spm_base.at[pl.ds(sidx * LCAP + k * STRIP, STRIP)])
        pltpu.sync_copy(lp.at[pl.ds(k * STRIP, STRIP)],
                        spm_p.at[pl.ds(sidx * LCAP + k * STRIP, STRIP)])
        return 0

    lax.fori_loop(0, (cnt + STRIP - 1) // STRIP, pub, 0)
    plsc.subcore_barrier()
    pltpu.sync_copy(cnt_spm, counts_v)

    def get_count(s2):
        v = plsc.load_gather(counts_v, [s2 * L + zeros16])
        return jnp.max(v)

    neg1 = jnp.full((L,), -1, jnp.int32)

    def wini(k, _):
        wmap[pl.ds(k * L, L)] = neg1
        return 0

    lax.fori_loop(0, RCELLS // L, wini, 0)

    def p1_s2(s2, _):
        cnt2 = get_count(s2)

        def p1_blk(k, _):
            pltpu.sync_copy(spm_base.at[pl.ds(s2 * LCAP + k * STRIP, STRIP)],
                            strip_base)
            pltpu.sync_copy(spm_p.at[pl.ds(s2 * LCAP + k * STRIP, STRIP)],
                            strip_p)
            rem = cnt2 - k * STRIP

            def p1_vec(j, _):
                bse = strip_base[pl.ds(j * L, L)]
                pp = strip_p[pl.ds(j * L, L)]
                cell = bse & (PLANE - 1)
                m = ((j * L + iota) < rem) & ((cell >> 13) == rid)
                key = jnp.where(m, cell, -1 - iota)
                dead = jnp.zeros((L,), jnp.bool_)
                for k2 in range(1, L):
                    perm = (iota + k2) & (L - 1)
                    keyr = jnp.take_along_axis(key, perm, axis=0,
                                               mode="promise_in_bounds")
                    dead = dead | ((keyr == key) & (perm > iota))
                plsc.store_scatter(wmap, [cell & (RCELLS - 1)], pp,
                                   mask=m & jnp.logical_not(dead))
                return 0

            lax.fori_loop(0, (jnp.minimum(rem, STRIP) + L - 1) // L, p1_vec, 0)
            return 0

        lax.fori_loop(0, (cnt2 + STRIP - 1) // STRIP, p1_blk, 0)
        return 0

    lax.fori_loop(0, NSUB, p1_s2, 0)

    def p2_s2(s2, _):
        cnt2 = get_count(s2)

        def p2_blk(k, _):
            pltpu.sync_copy(spm_base.at[pl.ds(s2 * LCAP + k * STRIP, STRIP)],
                            strip_base)
            pltpu.sync_copy(spm_p.at[pl.ds(s2 * LCAP + k * STRIP, STRIP)],
                            strip_p)
            rem = cnt2 - k * STRIP

            def p2_vec(j, R):
                bse = strip_base[pl.ds(j * L, L)]
                pp = strip_p[pl.ds(j * L, L)]
                cell = bse & (PLANE - 1)
                m = ((j * L + iota) < rem) & ((cell >> 13) == rid)
                win = plsc.load_gather(wmap, [cell & (RCELLS - 1)], mask=m)
                alive = m & (win == pp)
                plsc.store_compressed(base_sc.at[pl.ds(R, L)], bse, mask=alive)
                plsc.store_compressed(p_sc.at[pl.ds(R, L)], pp, mask=alive)
                return R + jnp.sum(alive.astype(jnp.int32))

            R = lax.fori_loop(0, (jnp.minimum(rem, STRIP) + L - 1) // L,
                              p2_vec, jnp.int32(0))

            def sct(q, _):
                i0 = jnp.full((L,), 2 * q, jnp.int32)
                i1 = jnp.minimum(i0 + 1, R - 1)
                pv0 = plsc.load_gather(p_sc, [i0])
                pv1 = plsc.load_gather(p_sc, [i1])
                for cb in range(C // L):
                    idxbuf[q, pl.ds(cb * L, L)] = pv0 * C + cb * L + iota
                    idxbuf[q, pl.ds(C + cb * L, L)] = pv1 * C + cb * L + iota
                pltpu.sync_copy(feats_h.at[idxbuf.at[q]], rows_v.at[q])
                bv0 = plsc.load_gather(base_sc, [i0])
                bv1 = plsc.load_gather(base_sc, [i1])
                for cb in range(C // L):
                    idxbuf[q, pl.ds(cb * L, L)] = bv0 + coff[cb]
                    idxbuf[q, pl.ds(C + cb * L, L)] = bv1 + coff[cb]
                pltpu.sync_copy(rows_v.at[q], canvas_flat.at[idxbuf.at[q]])
                return 0

            lax.fori_loop(0, (R + 1) // 2, sct, 0)
            return 0

        lax.fori_loop(0, (cnt2 + STRIP - 1) // STRIP, p2_blk, 0)
        return 0

    lax.fori_loop(0, NSUB, p2_s2, 0)


def _make_sc_scatter(P):
    CH = -(-((P + NSUB - 1) // NSUB) // 8) * 8
    G = (CH + L - 1) // L
    mesh = plsc.VectorSubcoreMesh(core_axis_name="c", subcore_axis_name="s")
    body = functools.partial(_sc_scatter_body, P=P, CH=CH, G=G)
    return pl.kernel(
        body,
        out_type=(),
        mesh=mesh,
        compiler_params=pltpu.CompilerParams(needs_layout_passes=False),
        scratch_types=[
            pltpu.VMEM((CH * 4,), jnp.int32),
            pltpu.VMEM((L,), jnp.int32),
            pltpu.VMEM((LCAP,), jnp.int32),
            pltpu.VMEM((LCAP,), jnp.int32),
            pltpu.VMEM((L,), jnp.int32),
            pltpu.VMEM((NSUB * L,), jnp.int32),
            pltpu.VMEM((RCELLS,), jnp.int32),
            pltpu.VMEM((STRIP,), jnp.int32),
            pltpu.VMEM((STRIP,), jnp.int32),
            pltpu.VMEM((SCAP,), jnp.int32),
            pltpu.VMEM((SCAP,), jnp.int32),
            pltpu.VMEM((SCAP // 2, 2 * C), jnp.float32),
            pltpu.VMEM((SCAP // 2, 2 * C), jnp.int32),
            pltpu.VMEM_SHARED((NSUB * L,), jnp.int32),
            pltpu.VMEM_SHARED((NSUB * LCAP,), jnp.int32),
            pltpu.VMEM_SHARED((NSUB * LCAP,), jnp.int32),
        ],
    )


def kernel(voxel_features, coords, batch_size):
    P = coords.shape[0]
    bound = jnp.minimum(jnp.asarray(batch_size, jnp.int32), NB)
    bound_vec = jnp.full((L,), bound, jnp.int32)
    ch = -(-((P + NSUB - 1) // NSUB) // 8) * 8
    coords4 = jnp.pad(coords.astype(jnp.int32),
                      ((0, ch * NSUB - P), (0, 1))).reshape(ch * NSUB * 4)
    feats_flat = voxel_features.reshape(P * C)
    canvas = _make_canvas()
    ref = jax.new_ref(canvas)
    _make_sc_scatter(P)(ref, coords4, feats_flat, bound_vec)
    return ref[...].reshape(NB, C, NX, NY)


# speedup vs baseline: 6.8137x; 1.2372x over previous
import functools

import jax
import jax.numpy as jnp
from jax import lax
from jax.experimental import pallas as pl
from jax.experimental.pallas import tpu as pltpu
from jax.experimental.pallas import tpu_sc as plsc

C = 64
NX = 512
NY = 512
NB = 4
NROWS = NB * C * NX
L = 16
NSUB = 16
NRG = NB * NX
GSUB = NRG // 32
LCAP = 4096
STRIP = 256


def _zero_body(o_ref):
    o_ref[...] = jnp.zeros_like(o_ref)


def _make_canvas():
    return pl.pallas_call(
        _zero_body,
        out_shape=jax.ShapeDtypeStruct((NB, C, NX, NY), jnp.float32),
        grid=(NB, C // 8),
        out_specs=pl.BlockSpec((1, 8, NX, NY), lambda b, c: (b, c, 0, 0)),
    )()


def _sc_scatter_body(canvas, coords_h, feats_h, bound_h,
                     coords_v, bound_v, lbase, lp, cnt16, counts_v, pres,
                     strip_base, strip_p, staging, gidx, rowbuf,
                     cnt_spm, spm_base, spm_p, P, CH, G):
    cidx = lax.axis_index("c")
    sidx = lax.axis_index("s")
    iota = lax.broadcasted_iota(jnp.int32, (L,), 0)
    zeros16 = jnp.zeros((L,), jnp.int32)
    zerosf = jnp.zeros((L,), jnp.float32)
    rid = cidx * NSUB + sidx
    cview = canvas.reshape(NROWS, NY)

    pltpu.sync_copy(bound_h, bound_v)
    pltpu.sync_copy(coords_h.at[pl.ds(sidx * CH * 4, CH * 4)], coords_v)
    bvec = bound_v[...]

    def scan_body(g, cnt):
        pidx = g * L + iota
        valid = (pidx < CH) & ((sidx * CH + pidx) < P)
        b = plsc.load_gather(coords_v, [pidx * 4], mask=valid)
        x = plsc.load_gather(coords_v, [pidx * 4 + 1], mask=valid)
        y = plsc.load_gather(coords_v, [pidx * 4 + 2], mask=valid)
        rg = b * NX + x
        acc = valid & (b >= 0) & (b < bvec) & ((rg >> 10) == cidx)
        base = (rg << 9) | y
        pg = sidx * CH + pidx
        plsc.store_compressed(lbase.at[pl.ds(cnt, L)], base, mask=acc)
        plsc.store_compressed(lp.at[pl.ds(cnt, L)], pg, mask=acc)
        return cnt + jnp.sum(acc.astype(jnp.int32))

    cnt = lax.fori_loop(0, G, scan_body, jnp.int32(0))

    cnt16[...] = jnp.full((L,), cnt, jnp.int32)
    pltpu.sync_copy(cnt16, cnt_spm.at[pl.ds(sidx * L, L)])

    def pub(k, _):
        pltpu.sync_copy(lbase.at[pl.ds(k * STRIP, STRIP)],
                        spm_base.at[pl.ds(sidx * LCAP + k * STRIP, STRIP)])
        pltpu.sync_copy(lp.at[pl.ds(k * STRIP, STRIP)],
                        spm_p.at[pl.ds(sidx * LCAP + k * STRIP, STRIP)])
        return 0

    lax.fori_loop(0, (cnt + STRIP - 1) // STRIP, pub, 0)
    plsc.subcore_barrier()
    pltpu.sync_copy(cnt_spm, counts_v)

    def get_count(s2):
        v = plsc.load_gather(counts_v, [s2 * L + zeros16])
        return jnp.max(v)

    for t in range(GSUB // L):
        pres[pl.ds(t * L, L)] = zeros16
    ones16 = zeros16 + 1

    def strips(per_vec):
        def s2_body(s2, carry):
            cnt2 = get_count(s2)

            def blk(k, carry):
                pltpu.sync_copy(
                    spm_base.at[pl.ds(s2 * LCAP + k * STRIP, STRIP)],
                    strip_base)
                pltpu.sync_copy(
                    spm_p.at[pl.ds(s2 * LCAP + k * STRIP, STRIP)], strip_p)
                rem = cnt2 - k * STRIP

                def vec(j, carry):
                    bse = strip_base[pl.ds(j * L, L)]
                    pp = strip_p[pl.ds(j * L, L)]
                    lv = (j * L + iota) < rem
                    return per_vec(bse, pp, lv, carry)

                return lax.fori_loop(
                    0, (jnp.minimum(rem, STRIP) + L - 1) // L, vec, carry)

            return lax.fori_loop(0, (cnt2 + STRIP - 1) // STRIP, blk, carry)

        return lax.fori_loop(0, NSUB, s2_body, 0)

    def pres_vec(bse, pp, lv, carry):
        m = lv & ((bse >> 15) == rid)
        plsc.store_scatter(pres, [(bse >> 9) & (GSUB - 1)], ones16, mask=m)
        return carry

    strips(pres_vec)

    def zst(r, _):
        def zcol(t, _):
            staging[r, pl.ds(t * L, L)] = zerosf
            return 0
        lax.fori_loop(0, NY // L, zcol, 0)
        return 0

    lax.fori_loop(0, C, zst, 0)

    def run_group(g, fill):
        grow = rid * GSUB + g

        def member_vec(bse, pp, lv, carry):
            m = lv & ((bse >> 9) == grow)

            def one(_, m):
                lane = plsc.all_reduce_ffs(m)
                yv = jnp.take_along_axis(bse & (NY - 1), lane, axis=0,
                                         mode="promise_in_bounds")
                if fill:
                    pv = jnp.take_along_axis(pp, lane, axis=0,
                                             mode="promise_in_bounds")
                    for cb in range(C // L):
                        gidx[0, pl.ds(cb * L, L)] = pv * C + cb * L + iota
                    pltpu.sync_copy(feats_h.at[gidx.at[0]], rowbuf.at[0])
                for cb in range(C // L):
                    val = (rowbuf[0, pl.ds(cb * L, L)] if fill else zerosf)
                    plsc.store_scatter(staging, [cb * L + iota, yv], val)
                return m & (iota != lane)

            nm = jnp.sum(m.astype(jnp.int32))
            lax.fori_loop(0, nm, one, m)
            return carry

        strips(member_vec)
        if fill:
            b = grow >> 9
            x = grow & (NX - 1)
            for cb in range(C // L):
                gidx[0, pl.ds(cb * L, L)] = (b * (C * NX) + x
                                             + (cb * L + iota) * NX)
            pltpu.sync_copy(staging, cview.at[gidx.at[0]])

    def group_body(g, _):
        gp = plsc.load_gather(pres, [g + zeros16])

        @pl.when(jnp.max(gp) > 0)
        def _():
            run_group(g, True)
            run_group(g, False)

        return 0

    lax.fori_loop(0, GSUB, group_body, 0)


def _make_sc_scatter(P):
    CH = -(-((P + NSUB - 1) // NSUB) // 8) * 8
    G = (CH + L - 1) // L
    mesh = plsc.VectorSubcoreMesh(core_axis_name="c", subcore_axis_name="s")
    body = functools.partial(_sc_scatter_body, P=P, CH=CH, G=G)
    return pl.kernel(
        body,
        out_type=(),
        mesh=mesh,
        compiler_params=pltpu.CompilerParams(needs_layout_passes=False),
        scratch_types=[
            pltpu.VMEM((CH * 4,), jnp.int32),
            pltpu.VMEM((L,), jnp.int32),
            pltpu.VMEM((LCAP,), jnp.int32),
            pltpu.VMEM((LCAP,), jnp.int32),
            pltpu.VMEM((L,), jnp.int32),
            pltpu.VMEM((NSUB * L,), jnp.int32),
            pltpu.VMEM((GSUB,), jnp.int32),
            pltpu.VMEM((STRIP,), jnp.int32),
            pltpu.VMEM((STRIP,), jnp.int32),
            pltpu.VMEM((C, NY), jnp.float32),
            pltpu.VMEM((1, C), jnp.int32),
            pltpu.VMEM((1, C), jnp.float32),
            pltpu.VMEM_SHARED((NSUB * L,), jnp.int32),
            pltpu.VMEM_SHARED((NSUB * LCAP,), jnp.int32),
            pltpu.VMEM_SHARED((NSUB * LCAP,), jnp.int32),
        ],
    )


def kernel(voxel_features, coords, batch_size):
    P = coords.shape[0]
    bound = jnp.minimum(jnp.asarray(batch_size, jnp.int32), NB)
    bound_vec = jnp.full((L,), bound, jnp.int32)
    ch = -(-((P + NSUB - 1) // NSUB) // 8) * 8
    coords4 = jnp.pad(coords.astype(jnp.int32),
                      ((0, ch * NSUB - P), (0, 1))).reshape(ch * NSUB * 4)
    feats_flat = voxel_features.reshape(P * C)
    canvas = _make_canvas()
    ref = jax.new_ref(canvas)
    _make_sc_scatter(P)(ref, coords4, feats_flat, bound_vec)
    return ref[...]


# speedup vs baseline: 12.8681x; 1.8886x over previous
import functools

import jax
import jax.numpy as jnp
from jax import lax
from jax.experimental import pallas as pl
from jax.experimental.pallas import tpu as pltpu
from jax.experimental.pallas import tpu_sc as plsc

C = 64
NX = 512
NY = 512
NB = 4
NROWS = NB * C * NX
L = 16
NSUB = 16
NRG = NB * NX
GSUB = NRG // 32
LCAP = 4096
STRIP = 256
CAPR = 2048


def _zero_body(o_ref):
    o_ref[...] = jnp.zeros_like(o_ref)


def _make_canvas():
    return pl.pallas_call(
        _zero_body,
        out_shape=jax.ShapeDtypeStruct((NB, C, NX, NY), jnp.float32),
        grid=(NB, C // 8),
        out_specs=pl.BlockSpec((1, 8, NX, NY), lambda b, c: (b, c, 0, 0)),
    )()


def _sc_scatter_body(canvas, coords_h, feats_h, bound_h,
                     coords_v, bound_v, lbase, lp, cnt16, counts_v, pres,
                     strip_base, strip_p, cbase, cp, staging, gidx, rowtile,
                     cnt_spm, spm_base, spm_p, P, CH, G):
    cidx = lax.axis_index("c")
    sidx = lax.axis_index("s")
    iota = lax.broadcasted_iota(jnp.int32, (L,), 0)
    zeros16 = jnp.zeros((L,), jnp.int32)
    zerosf = jnp.zeros((L,), jnp.float32)
    rid = cidx * NSUB + sidx
    cview = canvas.reshape(NROWS, NY)

    pltpu.sync_copy(bound_h, bound_v)
    row0 = jnp.minimum(sidx * CH, P - CH)
    pltpu.sync_copy(coords_h.at[pl.ds(row0 * 3, CH * 3)], coords_v)
    bvec = bound_v[...]

    def scan_body(g, cnt):
        pidx = g * L + iota
        valid = (pidx < CH) & ((row0 + pidx) >= sidx * CH)
        b = plsc.load_gather(coords_v, [pidx * 3], mask=valid)
        x = plsc.load_gather(coords_v, [pidx * 3 + 1], mask=valid)
        y = plsc.load_gather(coords_v, [pidx * 3 + 2], mask=valid)
        rg = b * NX + x
        acc = valid & (b >= 0) & (b < bvec) & ((rg >> 10) == cidx)
        base = (rg << 9) | y
        pg = row0 + pidx
        plsc.store_compressed(lbase.at[pl.ds(cnt, L)], base, mask=acc)
        plsc.store_compressed(lp.at[pl.ds(cnt, L)], pg, mask=acc)
        return cnt + jnp.sum(acc.astype(jnp.int32))

    cnt = lax.fori_loop(0, G, scan_body, jnp.int32(0))

    cnt16[...] = jnp.full((L,), cnt, jnp.int32)
    pltpu.sync_copy(cnt16, cnt_spm.at[pl.ds(sidx * L, L)])

    def pub(k, _):
        pltpu.sync_copy(lbase.at[pl.ds(k * STRIP, STRIP)],
                        spm_base.at[pl.ds(sidx * LCAP + k * STRIP, STRIP)])
        pltpu.sync_copy(lp.at[pl.ds(k * STRIP, STRIP)],
                        spm_p.at[pl.ds(sidx * LCAP + k * STRIP, STRIP)])
        return 0

    lax.fori_loop(0, (cnt + STRIP - 1) // STRIP, pub, 0)
    plsc.subcore_barrier()
    pltpu.sync_copy(cnt_spm, counts_v)

    def get_count(s2):
        v = plsc.load_gather(counts_v, [s2 * L + zeros16])
        return jnp.max(v)

    def strips(per_vec, carry0):
        def s2_body(s2, carry):
            cnt2 = get_count(s2)

            def blk(k, carry):
                pltpu.sync_copy(
                    spm_base.at[pl.ds(s2 * LCAP + k * STRIP, STRIP)],
                    strip_base)
                pltpu.sync_copy(
                    spm_p.at[pl.ds(s2 * LCAP + k * STRIP, STRIP)], strip_p)
                rem = cnt2 - k * STRIP

                def vec(j, carry):
                    bse = strip_base[pl.ds(j * L, L)]
                    pp = strip_p[pl.ds(j * L, L)]
                    lv = (j * L + iota) < rem
                    return per_vec(bse, pp, lv, carry)

                return lax.fori_loop(
                    0, (jnp.minimum(rem, STRIP) + L - 1) // L, vec, carry)

            return lax.fori_loop(0, (cnt2 + STRIP - 1) // STRIP, blk, carry)

        return lax.fori_loop(0, NSUB, s2_body, carry0)

    for t in range(GSUB // L):
        pres[pl.ds(t * L, L)] = zeros16
    ones16 = zeros16 + 1

    def cache_vec(bse, pp, lv, rcnt):
        m = lv & ((bse >> 15) == rid)
        plsc.store_scatter(pres, [(bse >> 9) & (GSUB - 1)], ones16, mask=m)
        at = jnp.minimum(rcnt, CAPR)
        plsc.store_compressed(cbase.at[pl.ds(at, L)], bse, mask=m)
        plsc.store_compressed(cp.at[pl.ds(at, L)], pp, mask=m)
        return rcnt + jnp.sum(m.astype(jnp.int32))

    rcnt = strips(cache_vec, jnp.int32(0))
    in_cache = rcnt <= CAPR

    def zst(r, _):
        def zcol(t, _):
            staging[r, pl.ds(t * L, L)] = zerosf
            return 0
        lax.fori_loop(0, NY // L, zcol, 0)
        return 0

    lax.fori_loop(0, C, zst, 0)

    def member_visitor(grow, fill):
        def member_vec(bse, pp, lv, carry):
            m = lv & ((bse >> 9) == grow)

            def one(_, m):
                lane = plsc.all_reduce_ffs(m)
                yv = jnp.take_along_axis(bse & (NY - 1), lane, axis=0,
                                         mode="promise_in_bounds")
                if fill:
                    pv = jnp.take_along_axis(pp, lane, axis=0,
                                             mode="promise_in_bounds")
                    prow = jnp.max(pv)
                    pltpu.sync_copy(feats_h.at[pl.ds((prow >> 3) * 8, 8)],
                                    rowtile)
                    sub = prow & 7
                for cb in range(C // L):
                    val = (rowtile[sub, pl.ds(cb * L, L)] if fill else zerosf)
                    plsc.store_scatter(staging, [cb * L + iota, yv], val)
                return m & (iota != lane)

            nm = jnp.sum(m.astype(jnp.int32))
            lax.fori_loop(0, nm, one, m)
            return carry

        return member_vec

    def run_group_cached(g, fill):
        grow = rid * GSUB + g
        visit = member_visitor(grow, fill)

        def vec(j, _):
            bse = cbase[pl.ds(j * L, L)]
            pp = cp[pl.ds(j * L, L)]
            lv = (j * L + iota) < rcnt
            return visit(bse, pp, lv, 0)

        lax.fori_loop(0, (rcnt + L - 1) // L, vec, 0)

    def run_group_strips(g, fill):
        grow = rid * GSUB + g
        strips(member_visitor(grow, fill), 0)

    def flush_group(g):
        grow = rid * GSUB + g
        b = grow >> 9
        x = grow & (NX - 1)
        for cb in range(C // L):
            gidx[0, pl.ds(cb * L, L)] = (b * (C * NX) + x
                                         + (cb * L + iota) * NX)
        pltpu.sync_copy(staging, cview.at[gidx.at[0]])

    def group_body(g, _):
        gp = plsc.load_gather(pres, [g + zeros16])

        @pl.when((jnp.max(gp) > 0) & in_cache)
        def _():
            run_group_cached(g, True)
            flush_group(g)
            run_group_cached(g, False)

        @pl.when((jnp.max(gp) > 0) & jnp.logical_not(in_cache))
        def _():
            run_group_strips(g, True)
            flush_group(g)
            run_group_strips(g, False)

        return 0

    lax.fori_loop(0, GSUB, group_body, 0)


def _make_sc_scatter(P):
    CH = -(-((P + NSUB - 1) // NSUB) // 8) * 8
    G = (CH + L - 1) // L
    mesh = plsc.VectorSubcoreMesh(core_axis_name="c", subcore_axis_name="s")
    body = functools.partial(_sc_scatter_body, P=P, CH=CH, G=G)
    return pl.kernel(
        body,
        out_type=(),
        mesh=mesh,
        compiler_params=pltpu.CompilerParams(needs_layout_passes=False),
        scratch_types=[
            pltpu.VMEM((CH * 3,), jnp.int32),
            pltpu.VMEM((L,), jnp.int32),
            pltpu.VMEM((LCAP,), jnp.int32),
            pltpu.VMEM((LCAP,), jnp.int32),
            pltpu.VMEM((L,), jnp.int32),
            pltpu.VMEM((NSUB * L,), jnp.int32),
            pltpu.VMEM((GSUB,), jnp.int32),
            pltpu.VMEM((STRIP,), jnp.int32),
            pltpu.VMEM((STRIP,), jnp.int32),
            pltpu.VMEM((CAPR + L,), jnp.int32),
            pltpu.VMEM((CAPR + L,), jnp.int32),
            pltpu.VMEM((C, NY), jnp.float32),
            pltpu.VMEM((1, C), jnp.int32),
            pltpu.VMEM((8, C), jnp.float32),
            pltpu.VMEM_SHARED((NSUB * L,), jnp.int32),
            pltpu.VMEM_SHARED((NSUB * LCAP,), jnp.int32),
            pltpu.VMEM_SHARED((NSUB * LCAP,), jnp.int32),
        ],
    )


def kernel(voxel_features, coords, batch_size):
    P = coords.shape[0]
    bound = jnp.minimum(jnp.asarray(batch_size, jnp.int32), NB)
    bound_vec = jnp.full((L,), bound, jnp.int32)
    coords_flat = coords.astype(jnp.int32).reshape(P * 3)
    canvas = _make_canvas()
    ref = jax.new_ref(canvas)
    _make_sc_scatter(P)(ref, coords_flat, voxel_features, bound_vec)
    return ref[...]
